# Initial kernel scaffold; baseline (speedup 1.0000x reference)
import jax
import jax.numpy as jnp
from jax.experimental import pallas as pl


def kernel(x, table, W1, b1, W2, b2, W3, b3):
    raise NotImplementedError("write your pallas kernel here")


# speedup vs baseline: 5.2231x; 5.2231x over previous
import functools

import jax
import jax.numpy as jnp
from jax import lax
from jax.experimental import pallas as pl
from jax.experimental.pallas import tpu as pltpu
from jax.experimental.pallas import tpu_sc as plsc

B = 4096
L = 50
EMB = 64
NC = 2
NS = 16
NW = NC * NS
BPW = B // NW
LANES = 16
CGROUPS = EMB // LANES


def _pool_kernel(x_hbm, table_hbm, out_hbm, idx_v, buf, out_v, sem):
    wid = lax.axis_index("s") * NC + lax.axis_index("c")
    base = wid * BPW
    pltpu.sync_copy(x_hbm.at[pl.ds(base, BPW)], idx_v)

    inv_l = jnp.full((LANES,), 1.0 / L, dtype=jnp.float32)

    def body(b, carry):
        pltpu.async_copy(table_hbm.at[idx_v.at[b]], buf, sem).wait()

        def rbody(l, accs):
            return tuple(accs[c] + buf[l, pl.ds(c * LANES, LANES)]
                         for c in range(CGROUPS))

        accs = lax.fori_loop(
            0, L, rbody,
            tuple(jnp.zeros((LANES,), jnp.float32) for _ in range(CGROUPS)))
        for c in range(CGROUPS):
            out_v[b, pl.ds(c * LANES, LANES)] = accs[c] * inv_l
        return carry

    lax.fori_loop(0, BPW, body, 0)
    pltpu.sync_copy(out_v, out_hbm.at[pl.ds(base, BPW)])


@functools.partial(
    pl.kernel,
    mesh=plsc.VectorSubcoreMesh(core_axis_name="c", subcore_axis_name="s"),
    out_type=jax.ShapeDtypeStruct((B, EMB), jnp.float32),
    scratch_types=[
        pltpu.VMEM((BPW, L), jnp.int32),
        pltpu.VMEM((L, EMB), jnp.float32),
        pltpu.VMEM((BPW, EMB), jnp.float32),
        pltpu.SemaphoreType.DMA,
    ],
    compiler_params=pltpu.CompilerParams(use_tc_tiling_on_sc=False),
)
def _pool(x_hbm, table_hbm, out_hbm, idx_v, buf, out_v, sem):
    _pool_kernel(x_hbm, table_hbm, out_hbm, idx_v, buf, out_v, sem)


def _mlp_kernel(h_ref, w1_ref, b1_ref, w2_ref, b2_ref, w3_ref, b3_ref, o_ref):
    dn = (((1,), (1,)), ((), ()))
    h = h_ref[...]
    z = lax.dot_general(h, w1_ref[...], dn, preferred_element_type=jnp.float32)
    z = jnp.maximum(z + b1_ref[...], 0.0)
    z = lax.dot_general(z, w2_ref[...], dn, preferred_element_type=jnp.float32)
    z = jnp.maximum(z + b2_ref[...], 0.0)
    z = lax.dot_general(z, w3_ref[...], dn, preferred_element_type=jnp.float32)
    o_ref[...] = z + b3_ref[...]


def kernel(x, table, W1, b1, W2, b2, W3, b3):
    h = _pool(x, table)
    return pl.pallas_call(
        _mlp_kernel,
        out_shape=jax.ShapeDtypeStruct((B, EMB), jnp.float32),
    )(h, W1, b1.reshape(1, -1), W2, b2.reshape(1, -1), W3, b3.reshape(1, -1))


# speedup vs baseline: 7.1111x; 1.3615x over previous
import functools

import jax
import jax.numpy as jnp
from jax import lax
from jax.experimental import pallas as pl
from jax.experimental.pallas import tpu as pltpu
from jax.experimental.pallas import tpu_sc as plsc

B = 4096
L = 50
EMB = 64
NC = 2
NS = 16
NW = NC * NS
BPW = B // NW
LANES = 16
CGROUPS = EMB // LANES


NBUF = 2


def _pool_kernel(x_hbm, table_hbm, out_hbm, idx_v, bufs, out_v, sems):
    wid = lax.axis_index("s") * NC + lax.axis_index("c")
    base = wid * BPW
    pltpu.sync_copy(x_hbm.at[pl.ds(base, BPW)], idx_v)

    inv_l = jnp.full((LANES,), 1.0 / L, dtype=jnp.float32)

    for k in range(NBUF):
        pltpu.async_copy(table_hbm.at[idx_v.at[k]], bufs[k], sems[k])

    def body(g, carry):
        for k in range(NBUF):
            b = g * NBUF + k
            buf = bufs[k]
            pltpu.make_async_copy(table_hbm.at[idx_v.at[b]], buf, sems[k]
                                  ).wait()
            accs = [buf[0, pl.ds(c * LANES, LANES)] for c in range(CGROUPS)]
            for l in range(1, L):
                for c in range(CGROUPS):
                    accs[c] = accs[c] + buf[l, pl.ds(c * LANES, LANES)]

            @pl.when(b + NBUF < BPW)
            def _():
                pltpu.async_copy(table_hbm.at[idx_v.at[b + NBUF]], buf,
                                 sems[k])

            for c in range(CGROUPS):
                out_v[b, pl.ds(c * LANES, LANES)] = accs[c] * inv_l
        return carry

    lax.fori_loop(0, BPW // NBUF, body, 0)
    pltpu.sync_copy(out_v, out_hbm.at[pl.ds(base, BPW)])


@functools.partial(
    pl.kernel,
    mesh=plsc.VectorSubcoreMesh(core_axis_name="c", subcore_axis_name="s"),
    out_type=jax.ShapeDtypeStruct((B, EMB), jnp.float32),
    scratch_types=[
        pltpu.VMEM((BPW, L), jnp.int32),
        *[pltpu.VMEM((L, EMB), jnp.float32) for _ in range(NBUF)],
        pltpu.VMEM((BPW, EMB), jnp.float32),
        *[pltpu.SemaphoreType.DMA for _ in range(NBUF)],
    ],
    compiler_params=pltpu.CompilerParams(use_tc_tiling_on_sc=False),
)
def _pool(x_hbm, table_hbm, out_hbm, idx_v, *rest):
    bufs = list(rest[:NBUF])
    out_v = rest[NBUF]
    sems = list(rest[NBUF + 1:NBUF + 1 + NBUF])
    _pool_kernel(x_hbm, table_hbm, out_hbm, idx_v, bufs, out_v, sems)


def _mlp_kernel(h_ref, w1_ref, b1_ref, w2_ref, b2_ref, w3_ref, b3_ref, o_ref):
    dn = (((1,), (1,)), ((), ()))
    h = h_ref[...]
    z = lax.dot_general(h, w1_ref[...], dn, preferred_element_type=jnp.float32)
    z = jnp.maximum(z + b1_ref[...], 0.0)
    z = lax.dot_general(z, w2_ref[...], dn, preferred_element_type=jnp.float32)
    z = jnp.maximum(z + b2_ref[...], 0.0)
    z = lax.dot_general(z, w3_ref[...], dn, preferred_element_type=jnp.float32)
    o_ref[...] = z + b3_ref[...]


def kernel(x, table, W1, b1, W2, b2, W3, b3):
    h = _pool(x, table)
    return pl.pallas_call(
        _mlp_kernel,
        out_shape=jax.ShapeDtypeStruct((B, EMB), jnp.float32),
    )(h, W1, b1.reshape(1, -1), W2, b2.reshape(1, -1), W3, b3.reshape(1, -1))
